# baseline (speedup 1.0000x reference)
import functools

import jax
import jax.numpy as jnp
from jax import lax
from jax.experimental import pallas as pl
from jax.experimental.pallas import tpu as pltpu
from jax.experimental.pallas import tpu_sc as plsc

N = 10000
E = 320000
D = 128
H = 64
EIG_K = 4
AVG_D_LOG = 3.4965
EPS = 1e-5

NPT = 320
NPAD = 10240
CH = 800
NCHUNK = E // CH
BB = 32
SELCAP = 256
FLUSH = 192


def _node_mm_body(h_ref, wa_ref, wb_ref, pb_ref, alo, ahi, blo, bhi):
    hb = h_ref[...]
    a = jnp.dot(hb, wa_ref[...], preferred_element_type=jnp.float32)
    b = jnp.dot(hb, wb_ref[...], preferred_element_type=jnp.float32) + pb_ref[...]
    alo[...] = a[:, :H]
    ahi[...] = a[:, H:]
    blo[...] = b[:, :H]
    bhi[...] = b[:, H:]


def _node_mm(h, wa, wb, pb):
    blk = 1000
    return pl.pallas_call(
        _node_mm_body,
        grid=(N // blk,),
        in_specs=[
            pl.BlockSpec((blk, D), lambda i: (i, 0)),
            pl.BlockSpec((D, D), lambda i: (0, 0)),
            pl.BlockSpec((D, D), lambda i: (0, 0)),
            pl.BlockSpec((1, D), lambda i: (0, 0)),
        ],
        out_specs=[pl.BlockSpec((blk, H), lambda i: (i, 0))] * 4,
        out_shape=[jax.ShapeDtypeStruct((N, H), jnp.float32)] * 4,
    )(h, wa, wb, pb)


def _edge_mm_body(e_ref, we_ref, glo, ghi):
    g = jnp.dot(e_ref[...], we_ref[...], preferred_element_type=jnp.float32)
    glo[...] = g[:, :H]
    ghi[...] = g[:, H:]


def _edge_mm(e, we):
    blk = 4000
    return pl.pallas_call(
        _edge_mm_body,
        grid=(E // blk,),
        in_specs=[
            pl.BlockSpec((blk, 16), lambda i: (i, 0)),
            pl.BlockSpec((16, D), lambda i: (0, 0)),
        ],
        out_specs=[pl.BlockSpec((blk, H), lambda i: (i, 0))] * 2,
        out_shape=[jax.ShapeDtypeStruct((E, H), jnp.float32)] * 2,
    )(e, we)


_info = plsc.get_sparse_core_info()
_NC, _NS = _info.num_cores, _info.num_subcores


def _sc_agg_body(hA, hB, g, src_hbm, dst_hbm, eig1_hbm,
                 sum_out, sq_out, wef_out, max_out, min_out, deg_out, wsum_out,
                 eig1_v, srcbuf, dstbuf, sel_eid, sel_src, sel_dst,
                 abuf, bbuf, gbuf, wbuf,
                 acc_sum, acc_sq, acc_wef, acc_max, acc_min, acc_deg, acc_wsum,
                 sem, sem2):
    cid = lax.axis_index("c")
    sid = lax.axis_index("s")
    wid = sid * _NC + cid
    lo = wid * NPT
    hi = lo + NPT

    iota16 = lax.iota(jnp.int32, 16)
    zero16 = jnp.zeros((16,), jnp.float32)
    ones16 = jnp.ones((16,), jnp.float32)
    ninf16 = jnp.full((16,), -3.0e38, jnp.float32)
    pinf16 = jnp.full((16,), 3.0e38, jnp.float32)
    zi16 = jnp.zeros((16,), jnp.int32)

    def init_acc(i, c):
        idx = i * 16 + iota16
        plsc.store_scatter(acc_sum, [idx], zero16)
        plsc.store_scatter(acc_sq, [idx], zero16)
        plsc.store_scatter(acc_wef, [idx], zero16)
        plsc.store_scatter(acc_max, [idx], ninf16)
        plsc.store_scatter(acc_min, [idx], pinf16)
        return c
    lax.fori_loop(0, NPT * H // 16, init_acc, 0)

    def init_dw(i, c):
        idx = i * 16 + iota16
        plsc.store_scatter(acc_deg, [idx], zero16)
        plsc.store_scatter(acc_wsum, [idx], zero16)
        return c
    lax.fori_loop(0, NPT // 16, init_dw, 0)

    def init_sel(i, c):
        idx = i * 16 + iota16
        plsc.store_scatter(sel_eid, [idx], zi16)
        plsc.store_scatter(sel_src, [idx], zi16)
        plsc.store_scatter(sel_dst, [idx], zi16)
        return c
    lax.fori_loop(0, SELCAP // 16, init_sel, 0)

    pltpu.sync_copy(eig1_hbm, eig1_v)

    def process_sel(total):
        nblk = (total + (BB - 1)) >> 5

        def issue_blk(b):
            pob = (b % 2) * BB
            pltpu.async_copy(hA.at[sel_src.at[pl.ds(b * BB, BB)]],
                             abuf.at[pl.ds(pob, BB)], sem)
            pltpu.async_copy(hB.at[sel_dst.at[pl.ds(b * BB, BB)]],
                             bbuf.at[pl.ds(pob, BB)], sem)
            pltpu.async_copy(g.at[sel_eid.at[pl.ds(b * BB, BB)]],
                             gbuf.at[pl.ds(pob, BB)], sem)

        @pl.when(nblk > 0)
        def _prime():
            issue_blk(0)

        def blk_body(b, bc):
            boff = b * BB
            pob = (b % 2) * BB
            pltpu.make_async_copy(hA.at[sel_src.at[pl.ds(boff, BB)]],
                                  abuf.at[pl.ds(pob, BB)], sem).wait()
            pltpu.make_async_copy(hB.at[sel_dst.at[pl.ds(boff, BB)]],
                                  bbuf.at[pl.ds(pob, BB)], sem).wait()
            pltpu.make_async_copy(g.at[sel_eid.at[pl.ds(boff, BB)]],
                                  gbuf.at[pl.ds(pob, BB)], sem).wait()

            @pl.when(b + 1 < nblk)
            def _next():
                issue_blk(b + 1)

            for j in range(BB // 16):
                svv = sel_src[pl.ds(boff + j * 16, 16)]
                dvv = sel_dst[pl.ds(boff + j * 16, 16)]
                es = plsc.load_gather(eig1_v, [svv])
                ed = plsc.load_gather(eig1_v, [dvv])
                wv = jnp.abs(es - ed)
                wbuf[pl.ds(j * 16, 16)] = wv
                live = (boff + j * 16 + iota16) < total
                dl = dvv - lo
                plsc.addupdate_scatter(acc_deg, [dl], ones16, mask=live)
                plsc.addupdate_scatter(acc_wsum, [dl], wv, mask=live)

            nrem = jnp.minimum(total - boff, BB)

            def edge_body(i, ec):
                dl = sel_dst[pl.ds(boff + i, 16)][0] - lo
                wi = wbuf[pl.ds(i, 16)][0]
                abase = dl * H
                for v in range(H // 16):
                    av = abuf[pob + i, pl.ds(v * 16, 16)]
                    bv = bbuf[pob + i, pl.ds(v * 16, 16)]
                    gv = gbuf[pob + i, pl.ds(v * 16, 16)]
                    ef = av + bv + gv
                    idxv = abase + v * 16 + iota16
                    plsc.addupdate_scatter(acc_sum, [idxv], ef)
                    plsc.addupdate_scatter(acc_sq, [idxv], ef * ef)
                    plsc.addupdate_scatter(acc_wef, [idxv], wi * ef)
                    m0 = plsc.load_gather(acc_max, [idxv])
                    plsc.store_scatter(acc_max, [idxv], jnp.maximum(m0, ef))
                    n0 = plsc.load_gather(acc_min, [idxv])
                    plsc.store_scatter(acc_min, [idxv], jnp.minimum(n0, ef))
                return ec
            lax.fori_loop(0, nrem, edge_body, 0)
            return bc
        lax.fori_loop(0, nblk, blk_body, 0)

    def issue_scan(c):
        par = (c % 2) * CH
        ca = pltpu.async_copy(src_hbm.at[pl.ds(c * CH, CH)],
                              srcbuf.at[pl.ds(par, CH)], sem2)
        cb = pltpu.async_copy(dst_hbm.at[pl.ds(c * CH, CH)],
                              dstbuf.at[pl.ds(par, CH)], sem2)
        return ca, cb

    issue_scan(0)

    def chunk_body(c, carry):
        base = c * CH
        par = (c % 2) * CH
        pltpu.make_async_copy(src_hbm.at[pl.ds(base, CH)],
                              srcbuf.at[pl.ds(par, CH)], sem2).wait()
        pltpu.make_async_copy(dst_hbm.at[pl.ds(base, CH)],
                              dstbuf.at[pl.ds(par, CH)], sem2).wait()

        @pl.when(c + 1 < NCHUNK)
        def _prefetch():
            issue_scan(c + 1)

        def scan_body(v, nsel):
            dv = dstbuf[pl.ds(par + v * 16, 16)]
            sv = srcbuf[pl.ds(par + v * 16, 16)]
            m = (dv >= lo) & (dv < hi)
            cnt = plsc.all_reduce_population_count(m)[0]
            plsc.store_compressed(sel_dst.at[pl.ds(nsel, 16)], dv, mask=m)
            plsc.store_compressed(sel_src.at[pl.ds(nsel, 16)], sv, mask=m)
            plsc.store_compressed(sel_eid.at[pl.ds(nsel, 16)],
                                  base + v * 16 + iota16, mask=m)
            nsel = nsel + cnt

            def do_flush(ns):
                process_sel(jnp.int32(FLUSH))
                for s in (sel_eid, sel_src, sel_dst):
                    vv = s[pl.ds(FLUSH, 16)]
                    s[pl.ds(0, 16)] = vv
                return ns - FLUSH

            return lax.cond(nsel >= FLUSH, do_flush, lambda ns: ns, nsel)
        nsel = lax.fori_loop(0, CH // 16, scan_body, jnp.int32(0), unroll=2)
        process_sel(nsel)
        return carry
    lax.fori_loop(0, NCHUNK, chunk_body, 0)

    pltpu.sync_copy(acc_sum, sum_out.at[pl.ds(lo * H, NPT * H)])
    pltpu.sync_copy(acc_sq, sq_out.at[pl.ds(lo * H, NPT * H)])
    pltpu.sync_copy(acc_wef, wef_out.at[pl.ds(lo * H, NPT * H)])
    pltpu.sync_copy(acc_max, max_out.at[pl.ds(lo * H, NPT * H)])
    pltpu.sync_copy(acc_min, min_out.at[pl.ds(lo * H, NPT * H)])
    pltpu.sync_copy(acc_deg, deg_out.at[pl.ds(lo, NPT)])
    pltpu.sync_copy(acc_wsum, wsum_out.at[pl.ds(lo, NPT)])


_sc_agg = functools.partial(
    pl.kernel,
    mesh=plsc.VectorSubcoreMesh(core_axis_name="c", subcore_axis_name="s"),
    compiler_params=pltpu.CompilerParams(use_tc_tiling_on_sc=False, needs_layout_passes=False),
    out_type=[jax.ShapeDtypeStruct((NPAD * H,), jnp.float32)] * 5
             + [jax.ShapeDtypeStruct((NPAD,), jnp.float32)] * 2,
    scratch_types=[
        pltpu.VMEM((NPAD,), jnp.float32),
        pltpu.VMEM((2 * CH,), jnp.int32),
        pltpu.VMEM((2 * CH,), jnp.int32),
        pltpu.VMEM((SELCAP,), jnp.int32),
        pltpu.VMEM((SELCAP,), jnp.int32),
        pltpu.VMEM((SELCAP,), jnp.int32),
        pltpu.VMEM((2 * BB, H), jnp.float32),
        pltpu.VMEM((2 * BB, H), jnp.float32),
        pltpu.VMEM((2 * BB, H), jnp.float32),
        pltpu.VMEM((BB + 16,), jnp.float32),
        pltpu.VMEM((NPT * H,), jnp.float32),
        pltpu.VMEM((NPT * H,), jnp.float32),
        pltpu.VMEM((NPT * H,), jnp.float32),
        pltpu.VMEM((NPT * H,), jnp.float32),
        pltpu.VMEM((NPT * H,), jnp.float32),
        pltpu.VMEM((NPT,), jnp.float32),
        pltpu.VMEM((NPT,), jnp.float32),
        pltpu.SemaphoreType.DMA,
        pltpu.SemaphoreType.DMA,
    ],
)(_sc_agg_body)


def _combine_body(h_ref, slo, shi, qlo, qhi, wlo, whi, xlo, xhi, nlo, nhi,
                  deg_ref, wsum_ref, snorm_ref,
                  p0_ref, pid_ref, pamp_ref, patt_ref, pb_ref,
                  hp_ref, ps_ref, pss_ref):
    deg = deg_ref[...]
    degc = jnp.maximum(deg, 1.0)
    has = deg > 0
    s = jnp.concatenate([slo[...], shi[...]], axis=1)
    mean = s / degc
    sq = jnp.concatenate([qlo[...], qhi[...]], axis=1) / degc
    std = jnp.sqrt(jax.nn.relu(sq - mean * mean) + EPS)
    mx = jnp.where(has, jnp.concatenate([xlo[...], xhi[...]], axis=1), 0.0)
    mn = jnp.where(has, jnp.concatenate([nlo[...], nhi[...]], axis=1), 0.0)
    dirv = jnp.concatenate([wlo[...], whi[...]], axis=1) / (wsum_ref[...] + 1e-8)
    agg = jnp.concatenate([mean, mx, mn, std, dirv], axis=1)
    logd = jnp.log(degc + 1.0)
    y = (jnp.dot(h_ref[...], p0_ref[...], preferred_element_type=jnp.float32)
         + jnp.dot(agg, pid_ref[...], preferred_element_type=jnp.float32)
         + (logd / AVG_D_LOG)
         * jnp.dot(agg, pamp_ref[...], preferred_element_type=jnp.float32)
         + (AVG_D_LOG / logd)
         * jnp.dot(agg, patt_ref[...], preferred_element_type=jnp.float32)
         + pb_ref[...])
    hp = y * snorm_ref[...]
    hp_ref[...] = hp
    ps_ref[...] = jnp.sum(hp, axis=0, keepdims=True)[None]
    pss_ref[...] = jnp.sum(hp * hp, axis=0, keepdims=True)[None]


def _combine(h, parts_lo, parts_hi, deg, wsum, snorm, p0, pid, pamp, patt, pb):
    blk = 1000
    nb = N // blk
    col = pl.BlockSpec((blk, H), lambda i: (i, 0))
    one = pl.BlockSpec((blk, 1), lambda i: (i, 0))
    slo, qlo, wlo, xlo, nlo = parts_lo
    shi, qhi, whi, xhi, nhi = parts_hi
    return pl.pallas_call(
        _combine_body,
        grid=(nb,),
        in_specs=[pl.BlockSpec((blk, D), lambda i: (i, 0)),
                  col, col, col, col, col, col, col, col, col, col,
                  one, one, one,
                  pl.BlockSpec((D, D), lambda i: (0, 0)),
                  pl.BlockSpec((5 * D, D), lambda i: (0, 0)),
                  pl.BlockSpec((5 * D, D), lambda i: (0, 0)),
                  pl.BlockSpec((5 * D, D), lambda i: (0, 0)),
                  pl.BlockSpec((1, D), lambda i: (0, 0))],
        out_specs=[pl.BlockSpec((blk, D), lambda i: (i, 0)),
                   pl.BlockSpec((1, 1, D), lambda i: (i, 0, 0)),
                   pl.BlockSpec((1, 1, D), lambda i: (i, 0, 0))],
        out_shape=[jax.ShapeDtypeStruct((N, D), jnp.float32),
                   jax.ShapeDtypeStruct((nb, 1, D), jnp.float32),
                   jax.ShapeDtypeStruct((nb, 1, D), jnp.float32)],
    )(h, slo, shi, qlo, qhi, wlo, whi, xlo, xhi, nlo, nhi,
      deg, wsum, snorm, p0, pid, pamp, patt, pb)


def _bn_body(hp_ref, ps_ref, pss_ref, gm_ref, bt_ref, o_ref):
    tot = jnp.sum(ps_ref[...][:, 0, :], axis=0, keepdims=True)
    tots = jnp.sum(pss_ref[...][:, 0, :], axis=0, keepdims=True)
    mu = tot / N
    var = tots / N - mu * mu
    o_ref[...] = ((hp_ref[...] - mu) * lax.rsqrt(var + EPS) * gm_ref[...]
                  + bt_ref[...])


def _bn(hp, ps, pss, gamma, beta):
    blk = 1000
    nb = N // blk
    return pl.pallas_call(
        _bn_body,
        grid=(nb,),
        in_specs=[pl.BlockSpec((blk, D), lambda i: (i, 0)),
                  pl.BlockSpec((nb, 1, D), lambda i: (0, 0, 0)),
                  pl.BlockSpec((nb, 1, D), lambda i: (0, 0, 0)),
                  pl.BlockSpec((1, D), lambda i: (0, 0)),
                  pl.BlockSpec((1, D), lambda i: (0, 0))],
        out_specs=pl.BlockSpec((blk, D), lambda i: (i, 0)),
        out_shape=jax.ShapeDtypeStruct((N, D), jnp.float32),
    )(hp, ps, pss, gamma, beta)


def kernel(h, e, snorm_n, eig, edge_index, pre_W, pre_b, post_W, post_b,
           bn_gamma, bn_beta):
    src = edge_index[0].astype(jnp.int32)
    dst = edge_index[1].astype(jnp.int32)

    hA_lo, hA_hi, hB_lo, hB_hi = _node_mm(
        h, pre_W[:D], pre_W[D:2 * D], pre_b.reshape(1, D))
    g_lo, g_hi = _edge_mm(e, pre_W[2 * D:])

    eig1 = jnp.pad(eig[:, 1], (0, NPAD - N))

    out_lo = _sc_agg(hA_lo, hB_lo, g_lo, src, dst, eig1)
    out_hi = _sc_agg(hA_hi, hB_hi, g_hi, src, dst, eig1)

    parts_lo = [a.reshape(NPAD, H)[:N] for a in out_lo[:5]]
    parts_hi = [a.reshape(NPAD, H)[:N] for a in out_hi[:5]]
    deg = out_lo[5][:N].reshape(N, 1)
    wsum = out_lo[6][:N].reshape(N, 1)

    hp, ps, pss = _combine(
        h, parts_lo, parts_hi, deg, wsum, snorm_n,
        post_W[:D], post_W[D:6 * D], post_W[6 * D:11 * D], post_W[11 * D:],
        post_b.reshape(1, D))
    return _bn(hp, ps, pss, bn_gamma.reshape(1, D), bn_beta.reshape(1, D))


# speedup vs baseline: 1.0492x; 1.0492x over previous
import functools

import jax
import jax.numpy as jnp
from jax import lax
from jax.experimental import pallas as pl
from jax.experimental.pallas import tpu as pltpu
from jax.experimental.pallas import tpu_sc as plsc

N = 10000
E = 320000
D = 128
H = 64
EIG_K = 4
AVG_D_LOG = 3.4965
EPS = 1e-5

NPT = 320
NPAD = 10240
CH = 800
NCHUNK = E // CH
BB = 32
SELCAP = 256
FLUSH = 192


def _node_mm_body(h_ref, wa_ref, wb_ref, pb_ref, alo, ahi, blo, bhi):
    hb = h_ref[...]
    a = jnp.dot(hb, wa_ref[...], preferred_element_type=jnp.float32)
    b = jnp.dot(hb, wb_ref[...], preferred_element_type=jnp.float32) + pb_ref[...]
    alo[...] = a[:, :H]
    ahi[...] = a[:, H:]
    blo[...] = b[:, :H]
    bhi[...] = b[:, H:]


def _node_mm(h, wa, wb, pb):
    blk = 1000
    return pl.pallas_call(
        _node_mm_body,
        grid=(N // blk,),
        in_specs=[
            pl.BlockSpec((blk, D), lambda i: (i, 0)),
            pl.BlockSpec((D, D), lambda i: (0, 0)),
            pl.BlockSpec((D, D), lambda i: (0, 0)),
            pl.BlockSpec((1, D), lambda i: (0, 0)),
        ],
        out_specs=[pl.BlockSpec((blk, H), lambda i: (i, 0))] * 4,
        out_shape=[jax.ShapeDtypeStruct((N, H), jnp.float32)] * 4,
    )(h, wa, wb, pb)


def _edge_mm_body(e_ref, we_ref, glo, ghi):
    g = jnp.dot(e_ref[...], we_ref[...], preferred_element_type=jnp.float32)
    glo[...] = g[:, :H]
    ghi[...] = g[:, H:]


def _edge_mm(e, we):
    blk = 4000
    return pl.pallas_call(
        _edge_mm_body,
        grid=(E // blk,),
        in_specs=[
            pl.BlockSpec((blk, 16), lambda i: (i, 0)),
            pl.BlockSpec((16, D), lambda i: (0, 0)),
        ],
        out_specs=[pl.BlockSpec((blk, H), lambda i: (i, 0))] * 2,
        out_shape=[jax.ShapeDtypeStruct((E, H), jnp.float32)] * 2,
    )(e, we)


_info = plsc.get_sparse_core_info()
_NC, _NS = _info.num_cores, _info.num_subcores


def _sc_agg_body(hA, hB, g, src_hbm, dst_hbm, eig1_hbm,
                 sum_out, sq_out, wef_out, max_out, min_out, deg_out, wsum_out,
                 eig1_v, srcbuf, dstbuf, sel_eid, sel_src, sel_dst,
                 abuf, bbuf, gbuf,
                 acc_sum, acc_sq, acc_wef, acc_max, acc_min, acc_deg, acc_wsum,
                 sem, sem2):
    cid = lax.axis_index("c")
    sid = lax.axis_index("s")
    wid = sid * _NC + cid
    lo = wid * NPT
    hi = lo + NPT

    iota16 = lax.iota(jnp.int32, 16)
    zero16 = jnp.zeros((16,), jnp.float32)
    ones16 = jnp.ones((16,), jnp.float32)
    ninf16 = jnp.full((16,), -3.0e38, jnp.float32)
    pinf16 = jnp.full((16,), 3.0e38, jnp.float32)
    zi16 = jnp.zeros((16,), jnp.int32)

    def init_acc(i, c):
        idx = i * 16 + iota16
        plsc.store_scatter(acc_sum, [idx], zero16)
        plsc.store_scatter(acc_sq, [idx], zero16)
        plsc.store_scatter(acc_wef, [idx], zero16)
        plsc.store_scatter(acc_max, [idx], ninf16)
        plsc.store_scatter(acc_min, [idx], pinf16)
        return c
    lax.fori_loop(0, NPT * H // 16, init_acc, 0)

    def init_dw(i, c):
        idx = i * 16 + iota16
        plsc.store_scatter(acc_deg, [idx], zero16)
        plsc.store_scatter(acc_wsum, [idx], zero16)
        return c
    lax.fori_loop(0, NPT // 16, init_dw, 0)

    def init_sel(i, c):
        idx = i * 16 + iota16
        plsc.store_scatter(sel_eid, [idx], zi16)
        plsc.store_scatter(sel_src, [idx], zi16)
        plsc.store_scatter(sel_dst, [idx], zi16)
        return c
    lax.fori_loop(0, SELCAP // 16, init_sel, 0)

    pltpu.sync_copy(eig1_hbm, eig1_v)

    def process_sel(total):
        nblk = (total + (BB - 1)) >> 5

        def issue_blk(b):
            pob = (b % 2) * BB
            pltpu.async_copy(hA.at[sel_src.at[pl.ds(b * BB, BB)]],
                             abuf.at[pl.ds(pob, BB)], sem)
            pltpu.async_copy(hB.at[sel_dst.at[pl.ds(b * BB, BB)]],
                             bbuf.at[pl.ds(pob, BB)], sem)
            pltpu.async_copy(g.at[sel_eid.at[pl.ds(b * BB, BB)]],
                             gbuf.at[pl.ds(pob, BB)], sem)

        @pl.when(nblk > 0)
        def _prime():
            issue_blk(0)

        def blk_body(b, bc):
            boff = b * BB
            pob = (b % 2) * BB
            pltpu.make_async_copy(hA.at[sel_src.at[pl.ds(boff, BB)]],
                                  abuf.at[pl.ds(pob, BB)], sem).wait()
            pltpu.make_async_copy(hB.at[sel_dst.at[pl.ds(boff, BB)]],
                                  bbuf.at[pl.ds(pob, BB)], sem).wait()
            pltpu.make_async_copy(g.at[sel_eid.at[pl.ds(boff, BB)]],
                                  gbuf.at[pl.ds(pob, BB)], sem).wait()

            @pl.when(b + 1 < nblk)
            def _next():
                issue_blk(b + 1)

            for j in range(BB // 16):
                goff = boff + j * 16
                svv = sel_src[pl.ds(goff, 16)]
                dvv = sel_dst[pl.ds(goff, 16)]
                es = plsc.load_gather(eig1_v, [svv])
                ed = plsc.load_gather(eig1_v, [dvv])
                wv = jnp.abs(es - ed)
                live = (goff + iota16) < total
                rloc = dvv - lo
                plsc.addupdate_scatter(acc_deg, [rloc], ones16, mask=live)
                plsc.addupdate_scatter(acc_wsum, [rloc], wv, mask=live)

                ngrp = jnp.clip(total - goff, 0, 16)

                def lane_body(i2, ec, j=j, rloc=rloc, wv=wv):
                    row = pob + j * 16 + i2
                    ind = zi16 + i2
                    rowd16 = rloc.at[ind].get(mode="promise_in_bounds")
                    wi16 = wv.at[ind].get(mode="promise_in_bounds")
                    base16 = rowd16 * H + iota16
                    for v in range(H // 16):
                        av = abuf[row, pl.ds(v * 16, 16)]
                        bv = bbuf[row, pl.ds(v * 16, 16)]
                        gv = gbuf[row, pl.ds(v * 16, 16)]
                        ef = av + bv + gv
                        idxv = base16 + v * 16
                        plsc.addupdate_scatter(acc_sum, [idxv], ef)
                        plsc.addupdate_scatter(acc_sq, [idxv], ef * ef)
                        plsc.addupdate_scatter(acc_wef, [idxv], wi16 * ef)
                        m0 = plsc.load_gather(acc_max, [idxv])
                        plsc.store_scatter(acc_max, [idxv], jnp.maximum(m0, ef))
                        n0 = plsc.load_gather(acc_min, [idxv])
                        plsc.store_scatter(acc_min, [idxv], jnp.minimum(n0, ef))
                    return ec
                lax.fori_loop(0, ngrp, lane_body, 0)
            return bc
        lax.fori_loop(0, nblk, blk_body, 0)

    def issue_scan(c):
        par = (c % 2) * CH
        ca = pltpu.async_copy(src_hbm.at[pl.ds(c * CH, CH)],
                              srcbuf.at[pl.ds(par, CH)], sem2)
        cb = pltpu.async_copy(dst_hbm.at[pl.ds(c * CH, CH)],
                              dstbuf.at[pl.ds(par, CH)], sem2)
        return ca, cb

    issue_scan(0)

    def chunk_body(c, carry):
        base = c * CH
        par = (c % 2) * CH
        pltpu.make_async_copy(src_hbm.at[pl.ds(base, CH)],
                              srcbuf.at[pl.ds(par, CH)], sem2).wait()
        pltpu.make_async_copy(dst_hbm.at[pl.ds(base, CH)],
                              dstbuf.at[pl.ds(par, CH)], sem2).wait()

        @pl.when(c + 1 < NCHUNK)
        def _prefetch():
            issue_scan(c + 1)

        def scan_body(v, nsel):
            dv = dstbuf[pl.ds(par + v * 16, 16)]
            sv = srcbuf[pl.ds(par + v * 16, 16)]
            m = (dv >= lo) & (dv < hi)
            cnt = plsc.all_reduce_population_count(m)[0]
            plsc.store_compressed(sel_dst.at[pl.ds(nsel, 16)], dv, mask=m)
            plsc.store_compressed(sel_src.at[pl.ds(nsel, 16)], sv, mask=m)
            plsc.store_compressed(sel_eid.at[pl.ds(nsel, 16)],
                                  base + v * 16 + iota16, mask=m)
            nsel = nsel + cnt

            def do_flush(ns):
                process_sel(jnp.int32(FLUSH))
                for s in (sel_eid, sel_src, sel_dst):
                    vv = s[pl.ds(FLUSH, 16)]
                    s[pl.ds(0, 16)] = vv
                return ns - FLUSH

            return lax.cond(nsel >= FLUSH, do_flush, lambda ns: ns, nsel)
        nsel = lax.fori_loop(0, CH // 16, scan_body, jnp.int32(0), unroll=2)
        process_sel(nsel)
        return carry
    lax.fori_loop(0, NCHUNK, chunk_body, 0)

    pltpu.sync_copy(acc_sum, sum_out.at[pl.ds(lo * H, NPT * H)])
    pltpu.sync_copy(acc_sq, sq_out.at[pl.ds(lo * H, NPT * H)])
    pltpu.sync_copy(acc_wef, wef_out.at[pl.ds(lo * H, NPT * H)])
    pltpu.sync_copy(acc_max, max_out.at[pl.ds(lo * H, NPT * H)])
    pltpu.sync_copy(acc_min, min_out.at[pl.ds(lo * H, NPT * H)])
    pltpu.sync_copy(acc_deg, deg_out.at[pl.ds(lo, NPT)])
    pltpu.sync_copy(acc_wsum, wsum_out.at[pl.ds(lo, NPT)])


_sc_agg = functools.partial(
    pl.kernel,
    mesh=plsc.VectorSubcoreMesh(core_axis_name="c", subcore_axis_name="s"),
    compiler_params=pltpu.CompilerParams(use_tc_tiling_on_sc=False, needs_layout_passes=False),
    out_type=[jax.ShapeDtypeStruct((NPAD * H,), jnp.float32)] * 5
             + [jax.ShapeDtypeStruct((NPAD,), jnp.float32)] * 2,
    scratch_types=[
        pltpu.VMEM((NPAD,), jnp.float32),
        pltpu.VMEM((2 * CH,), jnp.int32),
        pltpu.VMEM((2 * CH,), jnp.int32),
        pltpu.VMEM((SELCAP,), jnp.int32),
        pltpu.VMEM((SELCAP,), jnp.int32),
        pltpu.VMEM((SELCAP,), jnp.int32),
        pltpu.VMEM((2 * BB, H), jnp.float32),
        pltpu.VMEM((2 * BB, H), jnp.float32),
        pltpu.VMEM((2 * BB, H), jnp.float32),
        pltpu.VMEM((NPT * H,), jnp.float32),
        pltpu.VMEM((NPT * H,), jnp.float32),
        pltpu.VMEM((NPT * H,), jnp.float32),
        pltpu.VMEM((NPT * H,), jnp.float32),
        pltpu.VMEM((NPT * H,), jnp.float32),
        pltpu.VMEM((NPT,), jnp.float32),
        pltpu.VMEM((NPT,), jnp.float32),
        pltpu.SemaphoreType.DMA,
        pltpu.SemaphoreType.DMA,
    ],
)(_sc_agg_body)


def _combine_body(h_ref, slo, shi, qlo, qhi, wlo, whi, xlo, xhi, nlo, nhi,
                  deg_ref, wsum_ref, snorm_ref,
                  p0_ref, pid_ref, pamp_ref, patt_ref, pb_ref,
                  hp_ref, ps_ref, pss_ref):
    deg = deg_ref[...]
    degc = jnp.maximum(deg, 1.0)
    has = deg > 0
    s = jnp.concatenate([slo[...], shi[...]], axis=1)
    mean = s / degc
    sq = jnp.concatenate([qlo[...], qhi[...]], axis=1) / degc
    std = jnp.sqrt(jax.nn.relu(sq - mean * mean) + EPS)
    mx = jnp.where(has, jnp.concatenate([xlo[...], xhi[...]], axis=1), 0.0)
    mn = jnp.where(has, jnp.concatenate([nlo[...], nhi[...]], axis=1), 0.0)
    dirv = jnp.concatenate([wlo[...], whi[...]], axis=1) / (wsum_ref[...] + 1e-8)
    agg = jnp.concatenate([mean, mx, mn, std, dirv], axis=1)
    logd = jnp.log(degc + 1.0)
    y = (jnp.dot(h_ref[...], p0_ref[...], preferred_element_type=jnp.float32)
         + jnp.dot(agg, pid_ref[...], preferred_element_type=jnp.float32)
         + (logd / AVG_D_LOG)
         * jnp.dot(agg, pamp_ref[...], preferred_element_type=jnp.float32)
         + (AVG_D_LOG / logd)
         * jnp.dot(agg, patt_ref[...], preferred_element_type=jnp.float32)
         + pb_ref[...])
    hp = y * snorm_ref[...]
    hp_ref[...] = hp
    ps_ref[...] = jnp.sum(hp, axis=0, keepdims=True)[None]
    pss_ref[...] = jnp.sum(hp * hp, axis=0, keepdims=True)[None]


def _combine(h, parts_lo, parts_hi, deg, wsum, snorm, p0, pid, pamp, patt, pb):
    blk = 1000
    nb = N // blk
    col = pl.BlockSpec((blk, H), lambda i: (i, 0))
    one = pl.BlockSpec((blk, 1), lambda i: (i, 0))
    slo, qlo, wlo, xlo, nlo = parts_lo
    shi, qhi, whi, xhi, nhi = parts_hi
    return pl.pallas_call(
        _combine_body,
        grid=(nb,),
        in_specs=[pl.BlockSpec((blk, D), lambda i: (i, 0)),
                  col, col, col, col, col, col, col, col, col, col,
                  one, one, one,
                  pl.BlockSpec((D, D), lambda i: (0, 0)),
                  pl.BlockSpec((5 * D, D), lambda i: (0, 0)),
                  pl.BlockSpec((5 * D, D), lambda i: (0, 0)),
                  pl.BlockSpec((5 * D, D), lambda i: (0, 0)),
                  pl.BlockSpec((1, D), lambda i: (0, 0))],
        out_specs=[pl.BlockSpec((blk, D), lambda i: (i, 0)),
                   pl.BlockSpec((1, 1, D), lambda i: (i, 0, 0)),
                   pl.BlockSpec((1, 1, D), lambda i: (i, 0, 0))],
        out_shape=[jax.ShapeDtypeStruct((N, D), jnp.float32),
                   jax.ShapeDtypeStruct((nb, 1, D), jnp.float32),
                   jax.ShapeDtypeStruct((nb, 1, D), jnp.float32)],
    )(h, slo, shi, qlo, qhi, wlo, whi, xlo, xhi, nlo, nhi,
      deg, wsum, snorm, p0, pid, pamp, patt, pb)


def _bn_body(hp_ref, ps_ref, pss_ref, gm_ref, bt_ref, o_ref):
    tot = jnp.sum(ps_ref[...][:, 0, :], axis=0, keepdims=True)
    tots = jnp.sum(pss_ref[...][:, 0, :], axis=0, keepdims=True)
    mu = tot / N
    var = tots / N - mu * mu
    o_ref[...] = ((hp_ref[...] - mu) * lax.rsqrt(var + EPS) * gm_ref[...]
                  + bt_ref[...])


def _bn(hp, ps, pss, gamma, beta):
    blk = 1000
    nb = N // blk
    return pl.pallas_call(
        _bn_body,
        grid=(nb,),
        in_specs=[pl.BlockSpec((blk, D), lambda i: (i, 0)),
                  pl.BlockSpec((nb, 1, D), lambda i: (0, 0, 0)),
                  pl.BlockSpec((nb, 1, D), lambda i: (0, 0, 0)),
                  pl.BlockSpec((1, D), lambda i: (0, 0)),
                  pl.BlockSpec((1, D), lambda i: (0, 0))],
        out_specs=pl.BlockSpec((blk, D), lambda i: (i, 0)),
        out_shape=jax.ShapeDtypeStruct((N, D), jnp.float32),
    )(hp, ps, pss, gamma, beta)


def kernel(h, e, snorm_n, eig, edge_index, pre_W, pre_b, post_W, post_b,
           bn_gamma, bn_beta):
    src = edge_index[0].astype(jnp.int32)
    dst = edge_index[1].astype(jnp.int32)

    hA_lo, hA_hi, hB_lo, hB_hi = _node_mm(
        h, pre_W[:D], pre_W[D:2 * D], pre_b.reshape(1, D))
    g_lo, g_hi = _edge_mm(e, pre_W[2 * D:])

    eig1 = jnp.pad(eig[:, 1], (0, NPAD - N))

    out_lo = _sc_agg(hA_lo, hB_lo, g_lo, src, dst, eig1)
    out_hi = _sc_agg(hA_hi, hB_hi, g_hi, src, dst, eig1)

    parts_lo = [a.reshape(NPAD, H)[:N] for a in out_lo[:5]]
    parts_hi = [a.reshape(NPAD, H)[:N] for a in out_hi[:5]]
    deg = out_lo[5][:N].reshape(N, 1)
    wsum = out_lo[6][:N].reshape(N, 1)

    hp, ps, pss = _combine(
        h, parts_lo, parts_hi, deg, wsum, snorm_n,
        post_W[:D], post_W[D:6 * D], post_W[6 * D:11 * D], post_W[11 * D:],
        post_b.reshape(1, D))
    return _bn(hp, ps, pss, bn_gamma.reshape(1, D), bn_beta.reshape(1, D))


# speedup vs baseline: 1.1923x; 1.1363x over previous
import functools

import jax
import jax.numpy as jnp
from jax import lax
from jax.experimental import pallas as pl
from jax.experimental.pallas import tpu as pltpu
from jax.experimental.pallas import tpu_sc as plsc

N = 10000
E = 320000
D = 128
H = 64
EIG_K = 4
AVG_D_LOG = 3.4965
EPS = 1e-5

NPT = 320
NPAD = 10240
CH = 800
NCHUNK = E // CH
BB = 32
SELCAP = 288
FLUSH = 192


def _node_mm_body(h_ref, wa_ref, wb_ref, pb_ref, alo, ahi, blo, bhi):
    hb = h_ref[...]
    a = jnp.dot(hb, wa_ref[...], preferred_element_type=jnp.float32)
    b = jnp.dot(hb, wb_ref[...], preferred_element_type=jnp.float32) + pb_ref[...]
    alo[...] = a[:, :H]
    ahi[...] = a[:, H:]
    blo[...] = b[:, :H]
    bhi[...] = b[:, H:]


def _node_mm(h, wa, wb, pb):
    blk = 1000
    return pl.pallas_call(
        _node_mm_body,
        grid=(N // blk,),
        in_specs=[
            pl.BlockSpec((blk, D), lambda i: (i, 0)),
            pl.BlockSpec((D, D), lambda i: (0, 0)),
            pl.BlockSpec((D, D), lambda i: (0, 0)),
            pl.BlockSpec((1, D), lambda i: (0, 0)),
        ],
        out_specs=[pl.BlockSpec((blk, H), lambda i: (i, 0))] * 4,
        out_shape=[jax.ShapeDtypeStruct((N, H), jnp.float32)] * 4,
    )(h, wa, wb, pb)


def _edge_mm_body(e_ref, we_ref, glo, ghi):
    g = jnp.dot(e_ref[...], we_ref[...], preferred_element_type=jnp.float32)
    glo[...] = g[:, :H]
    ghi[...] = g[:, H:]


def _edge_mm(e, we):
    blk = 4000
    return pl.pallas_call(
        _edge_mm_body,
        grid=(E // blk,),
        in_specs=[
            pl.BlockSpec((blk, 16), lambda i: (i, 0)),
            pl.BlockSpec((16, D), lambda i: (0, 0)),
        ],
        out_specs=[pl.BlockSpec((blk, H), lambda i: (i, 0))] * 2,
        out_shape=[jax.ShapeDtypeStruct((E, H), jnp.float32)] * 2,
    )(e, we)


_info = plsc.get_sparse_core_info()
_NC, _NS = _info.num_cores, _info.num_subcores


def _sc_agg_body(hA, hB, g, src_hbm, dst_hbm, eig1_hbm,
                 sum_out, sq_out, wef_out, max_out, min_out, deg_out, wsum_out,
                 eig1_v, srcbuf, dstbuf, sel_eid, sel_src, sel_dst,
                 abuf, bbuf, gbuf,
                 acc_sum, acc_sq, acc_wef, acc_max, acc_min, acc_deg, acc_wsum,
                 sem, sem2):
    cid = lax.axis_index("c")
    sid = lax.axis_index("s")
    wid = sid * _NC + cid
    lo = wid * NPT
    hi = lo + NPT

    iota16 = lax.iota(jnp.int32, 16)
    zero16 = jnp.zeros((16,), jnp.float32)
    ones16 = jnp.ones((16,), jnp.float32)
    ninf16 = jnp.full((16,), -3.0e38, jnp.float32)
    pinf16 = jnp.full((16,), 3.0e38, jnp.float32)
    zi16 = jnp.zeros((16,), jnp.int32)

    def init_acc(i, c):
        idx = i * 16 + iota16
        plsc.store_scatter(acc_sum, [idx], zero16)
        plsc.store_scatter(acc_sq, [idx], zero16)
        plsc.store_scatter(acc_wef, [idx], zero16)
        plsc.store_scatter(acc_max, [idx], ninf16)
        plsc.store_scatter(acc_min, [idx], pinf16)
        return c
    lax.fori_loop(0, NPT * H // 16, init_acc, 0)

    def init_dw(i, c):
        idx = i * 16 + iota16
        plsc.store_scatter(acc_deg, [idx], zero16)
        plsc.store_scatter(acc_wsum, [idx], zero16)
        return c
    lax.fori_loop(0, NPT // 16, init_dw, 0)

    def init_sel(i, c):
        idx = i * 16 + iota16
        plsc.store_scatter(sel_eid, [idx], zi16)
        plsc.store_scatter(sel_src, [idx], zi16)
        plsc.store_scatter(sel_dst, [idx], zi16)
        return c
    lax.fori_loop(0, SELCAP // 16, init_sel, 0)

    pltpu.sync_copy(eig1_hbm, eig1_v)

    def process_sel(total):
        nblk = (total + (BB - 1)) >> 5

        def issue_blk(b):
            pob = (b % 2) * BB
            pltpu.async_copy(hA.at[sel_src.at[pl.ds(b * BB, BB)]],
                             abuf.at[pl.ds(pob, BB)], sem)
            pltpu.async_copy(hB.at[sel_dst.at[pl.ds(b * BB, BB)]],
                             bbuf.at[pl.ds(pob, BB)], sem)
            pltpu.async_copy(g.at[sel_eid.at[pl.ds(b * BB, BB)]],
                             gbuf.at[pl.ds(pob, BB)], sem)

        @pl.when(nblk > 0)
        def _prime():
            issue_blk(0)

        def blk_body(b, bc):
            boff = b * BB
            pob = (b % 2) * BB
            pltpu.make_async_copy(hA.at[sel_src.at[pl.ds(boff, BB)]],
                                  abuf.at[pl.ds(pob, BB)], sem).wait()
            pltpu.make_async_copy(hB.at[sel_dst.at[pl.ds(boff, BB)]],
                                  bbuf.at[pl.ds(pob, BB)], sem).wait()
            pltpu.make_async_copy(g.at[sel_eid.at[pl.ds(boff, BB)]],
                                  gbuf.at[pl.ds(pob, BB)], sem).wait()

            @pl.when(b + 1 < nblk)
            def _next():
                issue_blk(b + 1)

            for j in range(BB // 16):
                goff = boff + j * 16
                svv = sel_src[pl.ds(goff, 16)]
                dvv = sel_dst[pl.ds(goff, 16)]
                es = plsc.load_gather(eig1_v, [svv])
                ed = plsc.load_gather(eig1_v, [dvv])
                wv = jnp.abs(es - ed)
                live = (goff + iota16) < total
                rloc = dvv - lo
                plsc.addupdate_scatter(acc_deg, [rloc], ones16, mask=live)
                plsc.addupdate_scatter(acc_wsum, [rloc], wv, mask=live)

                ngrp = jnp.clip(total - goff, 0, 16)

                def lane_body(i2, ec, j=j, rloc=rloc, wv=wv):
                    row = pob + j * 16 + i2
                    ind = zi16 + i2
                    rowd16 = rloc.at[ind].get(mode="promise_in_bounds")
                    wi16 = wv.at[ind].get(mode="promise_in_bounds")
                    base16 = rowd16 * H + iota16
                    for v in range(H // 16):
                        av = abuf[row, pl.ds(v * 16, 16)]
                        bv = bbuf[row, pl.ds(v * 16, 16)]
                        gv = gbuf[row, pl.ds(v * 16, 16)]
                        ef = av + bv + gv
                        idxv = base16 + v * 16
                        plsc.addupdate_scatter(acc_sum, [idxv], ef)
                        plsc.addupdate_scatter(acc_sq, [idxv], ef * ef)
                        plsc.addupdate_scatter(acc_wef, [idxv], wi16 * ef)
                        m0 = plsc.load_gather(acc_max, [idxv])
                        plsc.store_scatter(acc_max, [idxv], jnp.maximum(m0, ef))
                        n0 = plsc.load_gather(acc_min, [idxv])
                        plsc.store_scatter(acc_min, [idxv], jnp.minimum(n0, ef))
                    return ec
                lax.fori_loop(0, ngrp, lane_body, 0)
            return bc
        lax.fori_loop(0, nblk, blk_body, 0)

    def issue_scan(c):
        par = (c % 2) * CH
        ca = pltpu.async_copy(src_hbm.at[pl.ds(c * CH, CH)],
                              srcbuf.at[pl.ds(par, CH)], sem2)
        cb = pltpu.async_copy(dst_hbm.at[pl.ds(c * CH, CH)],
                              dstbuf.at[pl.ds(par, CH)], sem2)
        return ca, cb

    issue_scan(0)

    def chunk_body(c, carry):
        base = c * CH
        par = (c % 2) * CH
        pltpu.make_async_copy(src_hbm.at[pl.ds(base, CH)],
                              srcbuf.at[pl.ds(par, CH)], sem2).wait()
        pltpu.make_async_copy(dst_hbm.at[pl.ds(base, CH)],
                              dstbuf.at[pl.ds(par, CH)], sem2).wait()

        @pl.when(c + 1 < NCHUNK)
        def _prefetch():
            issue_scan(c + 1)

        def scan_body(vo, nsel):
            for u in range(4):
                off = par + vo * 64 + u * 16
                dv = dstbuf[pl.ds(off, 16)]
                sv = srcbuf[pl.ds(off, 16)]
                m = (dv >= lo) & (dv < hi)
                cnt = plsc.all_reduce_population_count(m)[0]
                plsc.store_compressed(sel_dst.at[pl.ds(nsel, 16)], dv, mask=m)
                plsc.store_compressed(sel_src.at[pl.ds(nsel, 16)], sv, mask=m)
                plsc.store_compressed(sel_eid.at[pl.ds(nsel, 16)],
                                      base + vo * 64 + u * 16 + iota16, mask=m)
                nsel = nsel + cnt

            def do_flush(ns):
                process_sel(jnp.int32(FLUSH))
                for s in (sel_eid, sel_src, sel_dst):
                    for k in range(4):
                        vv = s[pl.ds(FLUSH + k * 16, 16)]
                        s[pl.ds(k * 16, 16)] = vv
                return ns - FLUSH

            return lax.cond(nsel >= FLUSH, do_flush, lambda ns: ns, nsel)
        nsel = lax.fori_loop(0, CH // 64, scan_body, jnp.int32(0))
        process_sel(nsel)
        return carry
    lax.fori_loop(0, NCHUNK, chunk_body, 0)

    pltpu.sync_copy(acc_sum, sum_out.at[pl.ds(lo * H, NPT * H)])
    pltpu.sync_copy(acc_sq, sq_out.at[pl.ds(lo * H, NPT * H)])
    pltpu.sync_copy(acc_wef, wef_out.at[pl.ds(lo * H, NPT * H)])
    pltpu.sync_copy(acc_max, max_out.at[pl.ds(lo * H, NPT * H)])
    pltpu.sync_copy(acc_min, min_out.at[pl.ds(lo * H, NPT * H)])
    pltpu.sync_copy(acc_deg, deg_out.at[pl.ds(lo, NPT)])
    pltpu.sync_copy(acc_wsum, wsum_out.at[pl.ds(lo, NPT)])


_sc_agg = functools.partial(
    pl.kernel,
    mesh=plsc.VectorSubcoreMesh(core_axis_name="c", subcore_axis_name="s"),
    compiler_params=pltpu.CompilerParams(use_tc_tiling_on_sc=False, needs_layout_passes=False),
    out_type=[jax.ShapeDtypeStruct((NPAD * H,), jnp.float32)] * 5
             + [jax.ShapeDtypeStruct((NPAD,), jnp.float32)] * 2,
    scratch_types=[
        pltpu.VMEM((NPAD,), jnp.float32),
        pltpu.VMEM((2 * CH,), jnp.int32),
        pltpu.VMEM((2 * CH,), jnp.int32),
        pltpu.VMEM((SELCAP,), jnp.int32),
        pltpu.VMEM((SELCAP,), jnp.int32),
        pltpu.VMEM((SELCAP,), jnp.int32),
        pltpu.VMEM((2 * BB, H), jnp.float32),
        pltpu.VMEM((2 * BB, H), jnp.float32),
        pltpu.VMEM((2 * BB, H), jnp.float32),
        pltpu.VMEM((NPT * H,), jnp.float32),
        pltpu.VMEM((NPT * H,), jnp.float32),
        pltpu.VMEM((NPT * H,), jnp.float32),
        pltpu.VMEM((NPT * H,), jnp.float32),
        pltpu.VMEM((NPT * H,), jnp.float32),
        pltpu.VMEM((NPT,), jnp.float32),
        pltpu.VMEM((NPT,), jnp.float32),
        pltpu.SemaphoreType.DMA,
        pltpu.SemaphoreType.DMA,
    ],
)(_sc_agg_body)


def _combine_body(h_ref, slo, shi, qlo, qhi, wlo, whi, xlo, xhi, nlo, nhi,
                  deg_ref, wsum_ref, snorm_ref,
                  p0_ref, pid_ref, pamp_ref, patt_ref, pb_ref,
                  hp_ref, ps_ref, pss_ref):
    deg = deg_ref[...]
    degc = jnp.maximum(deg, 1.0)
    has = deg > 0
    s = jnp.concatenate([slo[...], shi[...]], axis=1)
    mean = s / degc
    sq = jnp.concatenate([qlo[...], qhi[...]], axis=1) / degc
    std = jnp.sqrt(jax.nn.relu(sq - mean * mean) + EPS)
    mx = jnp.where(has, jnp.concatenate([xlo[...], xhi[...]], axis=1), 0.0)
    mn = jnp.where(has, jnp.concatenate([nlo[...], nhi[...]], axis=1), 0.0)
    dirv = jnp.concatenate([wlo[...], whi[...]], axis=1) / (wsum_ref[...] + 1e-8)
    agg = jnp.concatenate([mean, mx, mn, std, dirv], axis=1)
    logd = jnp.log(degc + 1.0)
    y = (jnp.dot(h_ref[...], p0_ref[...], preferred_element_type=jnp.float32)
         + jnp.dot(agg, pid_ref[...], preferred_element_type=jnp.float32)
         + (logd / AVG_D_LOG)
         * jnp.dot(agg, pamp_ref[...], preferred_element_type=jnp.float32)
         + (AVG_D_LOG / logd)
         * jnp.dot(agg, patt_ref[...], preferred_element_type=jnp.float32)
         + pb_ref[...])
    hp = y * snorm_ref[...]
    hp_ref[...] = hp
    ps_ref[...] = jnp.sum(hp, axis=0, keepdims=True)[None]
    pss_ref[...] = jnp.sum(hp * hp, axis=0, keepdims=True)[None]


def _combine(h, parts_lo, parts_hi, deg, wsum, snorm, p0, pid, pamp, patt, pb):
    blk = 1000
    nb = N // blk
    col = pl.BlockSpec((blk, H), lambda i: (i, 0))
    one = pl.BlockSpec((blk, 1), lambda i: (i, 0))
    slo, qlo, wlo, xlo, nlo = parts_lo
    shi, qhi, whi, xhi, nhi = parts_hi
    return pl.pallas_call(
        _combine_body,
        grid=(nb,),
        in_specs=[pl.BlockSpec((blk, D), lambda i: (i, 0)),
                  col, col, col, col, col, col, col, col, col, col,
                  one, one, one,
                  pl.BlockSpec((D, D), lambda i: (0, 0)),
                  pl.BlockSpec((5 * D, D), lambda i: (0, 0)),
                  pl.BlockSpec((5 * D, D), lambda i: (0, 0)),
                  pl.BlockSpec((5 * D, D), lambda i: (0, 0)),
                  pl.BlockSpec((1, D), lambda i: (0, 0))],
        out_specs=[pl.BlockSpec((blk, D), lambda i: (i, 0)),
                   pl.BlockSpec((1, 1, D), lambda i: (i, 0, 0)),
                   pl.BlockSpec((1, 1, D), lambda i: (i, 0, 0))],
        out_shape=[jax.ShapeDtypeStruct((N, D), jnp.float32),
                   jax.ShapeDtypeStruct((nb, 1, D), jnp.float32),
                   jax.ShapeDtypeStruct((nb, 1, D), jnp.float32)],
    )(h, slo, shi, qlo, qhi, wlo, whi, xlo, xhi, nlo, nhi,
      deg, wsum, snorm, p0, pid, pamp, patt, pb)


def _bn_body(hp_ref, ps_ref, pss_ref, gm_ref, bt_ref, o_ref):
    tot = jnp.sum(ps_ref[...][:, 0, :], axis=0, keepdims=True)
    tots = jnp.sum(pss_ref[...][:, 0, :], axis=0, keepdims=True)
    mu = tot / N
    var = tots / N - mu * mu
    o_ref[...] = ((hp_ref[...] - mu) * lax.rsqrt(var + EPS) * gm_ref[...]
                  + bt_ref[...])


def _bn(hp, ps, pss, gamma, beta):
    blk = 1000
    nb = N // blk
    return pl.pallas_call(
        _bn_body,
        grid=(nb,),
        in_specs=[pl.BlockSpec((blk, D), lambda i: (i, 0)),
                  pl.BlockSpec((nb, 1, D), lambda i: (0, 0, 0)),
                  pl.BlockSpec((nb, 1, D), lambda i: (0, 0, 0)),
                  pl.BlockSpec((1, D), lambda i: (0, 0)),
                  pl.BlockSpec((1, D), lambda i: (0, 0))],
        out_specs=pl.BlockSpec((blk, D), lambda i: (i, 0)),
        out_shape=jax.ShapeDtypeStruct((N, D), jnp.float32),
    )(hp, ps, pss, gamma, beta)


def kernel(h, e, snorm_n, eig, edge_index, pre_W, pre_b, post_W, post_b,
           bn_gamma, bn_beta):
    src = edge_index[0].astype(jnp.int32)
    dst = edge_index[1].astype(jnp.int32)

    hA_lo, hA_hi, hB_lo, hB_hi = _node_mm(
        h, pre_W[:D], pre_W[D:2 * D], pre_b.reshape(1, D))
    g_lo, g_hi = _edge_mm(e, pre_W[2 * D:])

    eig1 = jnp.pad(eig[:, 1], (0, NPAD - N))

    out_lo = _sc_agg(hA_lo, hB_lo, g_lo, src, dst, eig1)
    out_hi = _sc_agg(hA_hi, hB_hi, g_hi, src, dst, eig1)

    parts_lo = [a.reshape(NPAD, H)[:N] for a in out_lo[:5]]
    parts_hi = [a.reshape(NPAD, H)[:N] for a in out_hi[:5]]
    deg = out_lo[5][:N].reshape(N, 1)
    wsum = out_lo[6][:N].reshape(N, 1)

    hp, ps, pss = _combine(
        h, parts_lo, parts_hi, deg, wsum, snorm_n,
        post_W[:D], post_W[D:6 * D], post_W[6 * D:11 * D], post_W[11 * D:],
        post_b.reshape(1, D))
    return _bn(hp, ps, pss, bn_gamma.reshape(1, D), bn_beta.reshape(1, D))
